# Initial kernel scaffold; baseline (speedup 1.0000x reference)
import jax
import jax.numpy as jnp
from jax.experimental import pallas as pl


def kernel(node_attrs, node_feats, edge_attrs, edge_feats, edge_index, W1, W2, w_lin):
    raise NotImplementedError("write your pallas kernel here")


# speedup vs baseline: 2.3638x; 2.3638x over previous
import functools
import math

import jax
import jax.numpy as jnp
import numpy as np
from jax import lax
from jax.experimental import pallas as pl
from jax.experimental.pallas import tpu as pltpu
from jax.experimental.pallas import tpu_sc as plsc

N = 10000
E = 160000
NUM_ELEM = 10
EDGE_FEAT = 16
NODE_FEAT = 16
EDGE_ATTR = 4
OUT = 8
MLP_IN = EDGE_FEAT + 2 * NUM_ELEM

_NC = 2
_NS = 16
_NW = _NC * _NS
CHUNK = 128
CPT = 40
EPAD = _NW * CPT * CHUNK
SROW = 32
RROW = 16

BE = 2048


def _sc_gather(stab, rtab, sidx2, ridx2):
    mesh = plsc.VectorSubcoreMesh(core_axis_name="c", subcore_axis_name="s")

    @functools.partial(
        pl.kernel,
        out_type=(jax.ShapeDtypeStruct((EPAD, SROW), jnp.float32),
                  jax.ShapeDtypeStruct((EPAD, RROW), jnp.float32)),
        mesh=mesh,
        scratch_types=[
            pltpu.VMEM((CPT, CHUNK), jnp.int32),
            pltpu.VMEM((CPT, CHUNK), jnp.int32),
            pltpu.VMEM((CHUNK, SROW), jnp.float32),
            pltpu.VMEM((CHUNK, RROW), jnp.float32),
            pltpu.SemaphoreType.DMA,
        ],
        compiler_params=pltpu.CompilerParams(use_tc_tiling_on_sc=False),
    )
    def k(stab_h, rtab_h, sidx_h, ridx_h, gs_h, gr_h, sidx_v, ridx_v, sbuf, rbuf, sem):
        wid = lax.axis_index("s") * _NC + lax.axis_index("c")
        row0 = wid * CPT
        pltpu.sync_copy(sidx_h.at[pl.ds(row0, CPT)], sidx_v)
        pltpu.sync_copy(ridx_h.at[pl.ds(row0, CPT)], ridx_v)

        def body(j, carry):
            base = (row0 + j) * CHUNK
            pltpu.async_copy(stab_h.at[sidx_v.at[j]], sbuf, sem).wait()
            pltpu.sync_copy(sbuf, gs_h.at[pl.ds(base, CHUNK)])
            pltpu.async_copy(rtab_h.at[ridx_v.at[j]], rbuf, sem).wait()
            pltpu.sync_copy(rbuf, gr_h.at[pl.ds(base, CHUNK)])
            return carry

        lax.fori_loop(0, CPT, body, 0)

    return k(stab, rtab, sidx2, ridx2)


def _sc_scatter(mji, ridx2, zer):
    mesh = plsc.VectorSubcoreMesh(core_axis_name="c", subcore_axis_name="s")

    @functools.partial(
        pl.kernel,
        out_type=jax.ShapeDtypeStruct((_NC, N, OUT), jnp.float32),
        mesh=mesh,
        scratch_types=[
            pltpu.VMEM((CPT, CHUNK), jnp.int32),
            pltpu.VMEM((CHUNK, OUT), jnp.float32),
            pltpu.VMEM_SHARED((N, OUT), jnp.float32),
            pltpu.SemaphoreType.DMA,
        ],
        compiler_params=pltpu.CompilerParams(use_tc_tiling_on_sc=False),
    )
    def k(mji_h, ridx_h, zer_h, pout_h, ridx_v, mbuf, acc, sem):
        c = lax.axis_index("c")
        s = lax.axis_index("s")
        wid = c * _NS + s
        row0 = wid * CPT
        pltpu.sync_copy(ridx_h.at[pl.ds(row0, CPT)], ridx_v)

        @pl.when(s == 0)
        def _():
            pltpu.sync_copy(zer_h, acc)

        plsc.subcore_barrier()

        def body(j, carry):
            base = (row0 + j) * CHUNK
            pltpu.sync_copy(mji_h.at[pl.ds(base, CHUNK)], mbuf)
            pltpu.sync_copy(mbuf, acc.at[ridx_v.at[j]], add=True)
            return carry

        lax.fori_loop(0, CPT, body, 0)
        plsc.subcore_barrier()

        @pl.when(s == 0)
        def _():
            pltpu.sync_copy(acc, pout_h.at[c])

    return k(mji, ridx2, zer)


def _tc_compute(ef, ea, gs, gr, w1e, w1s, w1r, w2n, r1, r2, smat):
    def body(ef_r, ea_r, gs_r, gr_r, w1e_r, w1s_r, w1r_r, w2_r, r1_r, r2_r, s_r, out_r):
        dot = functools.partial(jnp.dot, preferred_element_type=jnp.float32)
        hpre = (dot(ef_r[...], w1e_r[...])
                + dot(gs_r[...], w1s_r[...])
                + dot(gr_r[...], w1r_r[...]))
        h = jnp.maximum(hpre, 0.0)
        t = dot(h, w2_r[...])
        orr = dot(gs_r[:, 0:16], r1_r[...]) * dot(ea_r[...], r2_r[...])
        out_r[...] = dot(t * orr, s_r[...])

    be = lambda d: pl.BlockSpec((BE, d), lambda i: (i, 0))
    full = lambda a: pl.BlockSpec(a.shape, lambda i: (0,) * a.ndim)
    return pl.pallas_call(
        body,
        grid=(EPAD // BE,),
        in_specs=[be(EDGE_FEAT), be(EDGE_ATTR), be(SROW), be(RROW),
                  full(w1e), full(w1s), full(w1r), full(w2n),
                  full(r1), full(r2), full(smat)],
        out_specs=be(OUT),
        out_shape=jax.ShapeDtypeStruct((EPAD, OUT), jnp.float32),
    )(ef, ea, gs, gr, w1e, w1s, w1r, w2n, r1, r2, smat)


def _combine(p):
    def body(p_r, o_r):
        o_r[...] = p_r[0] + p_r[1]

    return pl.pallas_call(
        body,
        out_shape=jax.ShapeDtypeStruct((N, OUT), jnp.float32),
    )(p)


def kernel(node_attrs, node_feats, edge_attrs, edge_feats, edge_index, W1, W2, w_lin):
    f32 = jnp.float32
    inv = 1.0 / math.sqrt(float(MLP_IN))
    w1n = W1 * inv
    w1e = w1n[0:EDGE_FEAT]
    w1s = jnp.concatenate(
        [jnp.zeros((NODE_FEAT, MLP_IN), f32),
         w1n[EDGE_FEAT:EDGE_FEAT + NUM_ELEM],
         jnp.zeros((SROW - NODE_FEAT - NUM_ELEM, MLP_IN), f32)], axis=0)
    w1r = jnp.concatenate(
        [w1n[EDGE_FEAT + NUM_ELEM:],
         jnp.zeros((RROW - NUM_ELEM, MLP_IN), f32)], axis=0)
    w2n = W2 * (math.sqrt(2.0) * inv)
    smat = jnp.tile(w_lin, (NODE_FEAT * EDGE_ATTR, 1)) * (
        1.0 / (math.sqrt(float(NODE_FEAT * EDGE_ATTR)) * math.sqrt(float(OUT))))
    r1 = jnp.asarray(np.repeat(np.eye(NODE_FEAT, dtype=np.float32),
                               EDGE_ATTR * OUT, axis=1))
    r2 = jnp.asarray(np.tile(np.repeat(np.eye(EDGE_ATTR, dtype=np.float32),
                                       OUT, axis=1), (1, NODE_FEAT)))

    stab = jnp.concatenate([node_feats, node_attrs, jnp.zeros((N, SROW - NODE_FEAT - NUM_ELEM), f32)], axis=1)
    rtab = jnp.concatenate([node_attrs, jnp.zeros((N, RROW - NUM_ELEM), f32)], axis=1)

    pad = EPAD - E
    sidx = jnp.pad(edge_index[0], (0, pad)).reshape(EPAD // CHUNK, CHUNK)
    ridx = jnp.pad(edge_index[1], (0, pad)).reshape(EPAD // CHUNK, CHUNK)
    efp = jnp.pad(edge_feats, ((0, pad), (0, 0)))
    eap = jnp.pad(edge_attrs, ((0, pad), (0, 0)))

    gs, gr = _sc_gather(stab, rtab, sidx, ridx)
    mji = _tc_compute(efp, eap, gs, gr, w1e, w1s, w1r, w2n, r1, r2, smat)
    zer = jnp.zeros((N, OUT), f32)
    p = _sc_scatter(mji, ridx, zer)
    return _combine(p)


# speedup vs baseline: 2.9951x; 1.2671x over previous
import functools
import math

import jax
import jax.numpy as jnp
import numpy as np
from jax import lax
from jax.experimental import pallas as pl
from jax.experimental.pallas import tpu as pltpu
from jax.experimental.pallas import tpu_sc as plsc

N = 10000
E = 160000
NUM_ELEM = 10
EDGE_FEAT = 16
NODE_FEAT = 16
EDGE_ATTR = 4
OUT = 8
MLP_IN = EDGE_FEAT + 2 * NUM_ELEM

_NC = 2
_NS = 16
_NW = _NC * _NS
CHUNK = 128
GCH = 5
NG = 8
GROWS = GCH * CHUNK
CPT = NG * GCH
EPT = CPT * CHUNK
EPAD = _NW * EPT
SROW = 32
RROW = 16

BE = 2048


def _sc_gather(stab, rtab, sidx2, ridx2):
    mesh = plsc.VectorSubcoreMesh(core_axis_name="c", subcore_axis_name="s")

    @functools.partial(
        pl.kernel,
        out_type=(jax.ShapeDtypeStruct((EPAD, SROW), jnp.float32),
                  jax.ShapeDtypeStruct((EPAD, RROW), jnp.float32)),
        mesh=mesh,
        scratch_types=[
            pltpu.VMEM((CPT, CHUNK), jnp.int32),
            pltpu.VMEM((CPT, CHUNK), jnp.int32),
            pltpu.VMEM((2, GROWS, SROW), jnp.float32),
            pltpu.VMEM((2, GROWS, RROW), jnp.float32),
            pltpu.SemaphoreType.DMA,
            pltpu.SemaphoreType.DMA,
            pltpu.SemaphoreType.DMA,
            pltpu.SemaphoreType.DMA,
        ],
        compiler_params=pltpu.CompilerParams(use_tc_tiling_on_sc=False),
    )
    def k(stab_h, rtab_h, sidx_h, ridx_h, gs_h, gr_h,
          sidx_v, ridx_v, sbuf, rbuf, gsem0, gsem1, wsem0, wsem1):
        wid = lax.axis_index("s") * _NC + lax.axis_index("c")
        crow0 = wid * CPT
        erow0 = wid * EPT
        pltpu.sync_copy(sidx_h.at[pl.ds(crow0, CPT)], sidx_v)
        pltpu.sync_copy(ridx_h.at[pl.ds(crow0, CPT)], ridx_v)

        gsems = (gsem0, gsem1)
        wsems = (wsem0, wsem1)

        def start_gathers(g, p):
            for c in range(GCH):
                j = g * GCH + c
                pltpu.async_copy(stab_h.at[sidx_v.at[j]],
                                 sbuf.at[p].at[pl.ds(c * CHUNK, CHUNK)], gsems[p])
                pltpu.async_copy(rtab_h.at[ridx_v.at[j]],
                                 rbuf.at[p].at[pl.ds(c * CHUNK, CHUNK)], gsems[p])

        def drain_gathers(p):
            pltpu.make_async_copy(stab_h.at[pl.ds(0, GROWS)], sbuf.at[p], gsems[p]).wait()
            pltpu.make_async_copy(rtab_h.at[pl.ds(0, GROWS)], rbuf.at[p], gsems[p]).wait()

        def start_wb(g, p):
            base = erow0 + g * GROWS
            pltpu.async_copy(sbuf.at[p], gs_h.at[pl.ds(base, GROWS)], wsems[p])
            pltpu.async_copy(rbuf.at[p], gr_h.at[pl.ds(base, GROWS)], wsems[p])

        def drain_wb(p):
            pltpu.make_async_copy(sbuf.at[p], gs_h.at[pl.ds(0, GROWS)], wsems[p]).wait()
            pltpu.make_async_copy(rbuf.at[p], gr_h.at[pl.ds(0, GROWS)], wsems[p]).wait()

        start_gathers(0, 0)

        def body(i, carry):
            g0 = 2 * i
            g1 = g0 + 1
            drain_gathers(0)
            start_wb(g0, 0)

            @pl.when(i > 0)
            def _():
                drain_wb(1)

            start_gathers(g1, 1)
            drain_gathers(1)
            start_wb(g1, 1)
            drain_wb(0)

            @pl.when(i < (NG // 2 - 1))
            def _():
                start_gathers(g0 + 2, 0)

            return carry

        lax.fori_loop(0, NG // 2, body, 0)
        drain_wb(1)

    return k(stab, rtab, sidx2, ridx2)


def _sc_scatter(mji, ridx2, zer):
    mesh = plsc.VectorSubcoreMesh(core_axis_name="c", subcore_axis_name="s")

    @functools.partial(
        pl.kernel,
        out_type=jax.ShapeDtypeStruct((_NC, N, OUT), jnp.float32),
        mesh=mesh,
        scratch_types=[
            pltpu.VMEM((CPT, CHUNK), jnp.int32),
            pltpu.VMEM((2, GROWS, OUT), jnp.float32),
            pltpu.VMEM_SHARED((N, OUT), jnp.float32),
            pltpu.SemaphoreType.DMA,
            pltpu.SemaphoreType.DMA,
        ],
        compiler_params=pltpu.CompilerParams(use_tc_tiling_on_sc=False),
    )
    def k(mji_h, ridx_h, zer_h, pout_h, ridx_v, mbuf, acc, lsem0, lsem1):
        c = lax.axis_index("c")
        s = lax.axis_index("s")
        wid = c * _NS + s
        crow0 = wid * CPT
        erow0 = wid * EPT
        pltpu.sync_copy(ridx_h.at[pl.ds(crow0, CPT)], ridx_v)

        @pl.when(s == 0)
        def _():
            pltpu.sync_copy(zer_h, acc)

        plsc.subcore_barrier()

        lsems = (lsem0, lsem1)

        def start_load(g, p):
            pltpu.async_copy(mji_h.at[pl.ds(erow0 + g * GROWS, GROWS)],
                             mbuf.at[p], lsems[p])

        def drain_load(p):
            pltpu.make_async_copy(mji_h.at[pl.ds(0, GROWS)], mbuf.at[p], lsems[p]).wait()

        def scatter_group(g, p):
            for cc in range(GCH):
                j = g * GCH + cc
                pltpu.sync_copy(mbuf.at[p].at[pl.ds(cc * CHUNK, CHUNK)],
                                acc.at[ridx_v.at[j]], add=True)

        start_load(0, 0)

        def body(i, carry):
            g0 = 2 * i
            g1 = g0 + 1
            start_load(g1, 1)
            drain_load(0)
            scatter_group(g0, 0)

            @pl.when(i < (NG // 2 - 1))
            def _():
                start_load(g0 + 2, 0)

            drain_load(1)
            scatter_group(g1, 1)
            return carry

        lax.fori_loop(0, NG // 2, body, 0)
        plsc.subcore_barrier()

        @pl.when(s == 0)
        def _():
            pltpu.sync_copy(acc, pout_h.at[c])

    return k(mji, ridx2, zer)


def _tc_compute(ef, ea, gs, gr, w1e, w1s, w1r, w2p, r1s, s2):
    def body(ef_r, ea_r, gs_r, gr_r, w1e_r, w1s_r, w1r_r, w2_r, r1_r, s2_r, out_r):
        dot = functools.partial(jnp.dot, preferred_element_type=jnp.float32)
        hpre = (dot(ef_r[...], w1e_r[...])
                + dot(gs_r[...], w1s_r[...])
                + dot(gr_r[...], w1r_r[...]))
        h = jnp.maximum(hpre, 0.0)
        t = dot(h, w2_r[...])
        ea_v = ea_r[...]
        v = (ea_v[:, 0:1] * t[:, 0:128]
             + ea_v[:, 1:2] * t[:, 128:256]
             + ea_v[:, 2:3] * t[:, 256:384]
             + ea_v[:, 3:4] * t[:, 384:512])
        x1e = dot(gs_r[:, 0:16], r1_r[...])
        out_r[...] = dot(v * x1e, s2_r[...])

    be = lambda d: pl.BlockSpec((BE, d), lambda i: (i, 0))
    full = lambda a: pl.BlockSpec(a.shape, lambda i: (0,) * a.ndim)
    return pl.pallas_call(
        body,
        grid=(EPAD // BE,),
        in_specs=[be(EDGE_FEAT), be(EDGE_ATTR), be(SROW), be(RROW),
                  full(w1e), full(w1s), full(w1r), full(w2p),
                  full(r1s), full(s2)],
        out_specs=be(OUT),
        out_shape=jax.ShapeDtypeStruct((EPAD, OUT), jnp.float32),
    )(ef, ea, gs, gr, w1e, w1s, w1r, w2p, r1s, s2)


def _combine(p):
    def body(p_r, o_r):
        o_r[...] = p_r[0] + p_r[1]

    return pl.pallas_call(
        body,
        out_shape=jax.ShapeDtypeStruct((N, OUT), jnp.float32),
    )(p)


def kernel(node_attrs, node_feats, edge_attrs, edge_feats, edge_index, W1, W2, w_lin):
    f32 = jnp.float32
    inv = 1.0 / math.sqrt(float(MLP_IN))
    w1n = W1 * inv
    w1e = w1n[0:EDGE_FEAT]
    w1s = jnp.concatenate(
        [jnp.zeros((NODE_FEAT, MLP_IN), f32),
         w1n[EDGE_FEAT:EDGE_FEAT + NUM_ELEM],
         jnp.zeros((SROW - NODE_FEAT - NUM_ELEM, MLP_IN), f32)], axis=0)
    w1r = jnp.concatenate(
        [w1n[EDGE_FEAT + NUM_ELEM:],
         jnp.zeros((RROW - NUM_ELEM, MLP_IN), f32)], axis=0)
    w2n = W2 * (math.sqrt(2.0) * inv)
    w2p = w2n.reshape(MLP_IN, NODE_FEAT, EDGE_ATTR, OUT).transpose(0, 2, 1, 3) \
             .reshape(MLP_IN, NODE_FEAT * EDGE_ATTR * OUT)
    s2 = jnp.tile(w_lin, (NODE_FEAT, 1)) * (
        1.0 / (math.sqrt(float(NODE_FEAT * EDGE_ATTR)) * math.sqrt(float(OUT))))
    r1s = jnp.asarray(np.repeat(np.eye(NODE_FEAT, dtype=np.float32), OUT, axis=1))

    stab = jnp.concatenate([node_feats, node_attrs, jnp.zeros((N, SROW - NODE_FEAT - NUM_ELEM), f32)], axis=1)
    rtab = jnp.concatenate([node_attrs, jnp.zeros((N, RROW - NUM_ELEM), f32)], axis=1)

    pad = EPAD - E
    sidx = jnp.pad(edge_index[0], (0, pad)).reshape(EPAD // CHUNK, CHUNK)
    ridx = jnp.pad(edge_index[1], (0, pad)).reshape(EPAD // CHUNK, CHUNK)
    efp = jnp.pad(edge_feats, ((0, pad), (0, 0)))
    eap = jnp.pad(edge_attrs, ((0, pad), (0, 0)))

    gs, gr = _sc_gather(stab, rtab, sidx, ridx)
    mji = _tc_compute(efp, eap, gs, gr, w1e, w1s, w1r, w2p, r1s, s2)
    zer = jnp.zeros((N, OUT), f32)
    p = _sc_scatter(mji, ridx, zer)
    return _combine(p)


# speedup vs baseline: 3.5487x; 1.1848x over previous
import functools
import math

import jax
import jax.numpy as jnp
import numpy as np
from jax import lax
from jax.experimental import pallas as pl
from jax.experimental.pallas import tpu as pltpu
from jax.experimental.pallas import tpu_sc as plsc

N = 10000
E = 160000
NUM_ELEM = 10
EDGE_FEAT = 16
NODE_FEAT = 16
EDGE_ATTR = 4
OUT = 8
MLP_IN = EDGE_FEAT + 2 * NUM_ELEM

_NC = 2
_NS = 16
_NW = _NC * _NS
CHUNK = 125
GCH = 5
NG = 8
GROWS = GCH * CHUNK
CPT = NG * GCH
EPT = CPT * CHUNK
SROW = 32
RROW = 16

BE = 2000


def _sc_gather(stab, rtab, sidx2, ridx2):
    mesh = plsc.VectorSubcoreMesh(core_axis_name="c", subcore_axis_name="s")

    @functools.partial(
        pl.kernel,
        out_type=(jax.ShapeDtypeStruct((E, SROW), jnp.float32),
                  jax.ShapeDtypeStruct((E, RROW), jnp.float32)),
        mesh=mesh,
        scratch_types=[
            pltpu.VMEM((CPT, CHUNK), jnp.int32),
            pltpu.VMEM((CPT, CHUNK), jnp.int32),
            pltpu.VMEM((2, GROWS, SROW), jnp.float32),
            pltpu.VMEM((2, GROWS, RROW), jnp.float32),
            pltpu.SemaphoreType.DMA,
            pltpu.SemaphoreType.DMA,
            pltpu.SemaphoreType.DMA,
            pltpu.SemaphoreType.DMA,
        ],
        compiler_params=pltpu.CompilerParams(use_tc_tiling_on_sc=False),
    )
    def k(stab_h, rtab_h, sidx_h, ridx_h, gs_h, gr_h,
          sidx_v, ridx_v, sbuf, rbuf, gsem0, gsem1, wsem0, wsem1):
        wid = lax.axis_index("s") * _NC + lax.axis_index("c")
        crow0 = wid * CPT
        erow0 = wid * EPT
        pltpu.sync_copy(sidx_h.at[pl.ds(crow0, CPT)], sidx_v)
        pltpu.sync_copy(ridx_h.at[pl.ds(crow0, CPT)], ridx_v)

        gsems = (gsem0, gsem1)
        wsems = (wsem0, wsem1)

        def start_gathers(g, p):
            for c in range(GCH):
                j = g * GCH + c
                pltpu.async_copy(stab_h.at[sidx_v.at[j]],
                                 sbuf.at[p].at[pl.ds(c * CHUNK, CHUNK)], gsems[p])
                pltpu.async_copy(rtab_h.at[ridx_v.at[j]],
                                 rbuf.at[p].at[pl.ds(c * CHUNK, CHUNK)], gsems[p])

        def drain_gathers(p):
            pltpu.make_async_copy(stab_h.at[pl.ds(0, GROWS)], sbuf.at[p], gsems[p]).wait()
            pltpu.make_async_copy(rtab_h.at[pl.ds(0, GROWS)], rbuf.at[p], gsems[p]).wait()

        def start_wb(g, p):
            base = erow0 + g * GROWS
            pltpu.async_copy(sbuf.at[p], gs_h.at[pl.ds(base, GROWS)], wsems[p])
            pltpu.async_copy(rbuf.at[p], gr_h.at[pl.ds(base, GROWS)], wsems[p])

        def drain_wb(p):
            pltpu.make_async_copy(sbuf.at[p], gs_h.at[pl.ds(0, GROWS)], wsems[p]).wait()
            pltpu.make_async_copy(rbuf.at[p], gr_h.at[pl.ds(0, GROWS)], wsems[p]).wait()

        start_gathers(0, 0)

        def body(i, carry):
            g0 = 2 * i
            g1 = g0 + 1
            drain_gathers(0)
            start_wb(g0, 0)

            @pl.when(i > 0)
            def _():
                drain_wb(1)

            start_gathers(g1, 1)
            drain_gathers(1)
            start_wb(g1, 1)
            drain_wb(0)

            @pl.when(i < (NG // 2 - 1))
            def _():
                start_gathers(g0 + 2, 0)

            return carry

        lax.fori_loop(0, NG // 2, body, 0)
        drain_wb(1)

    return k(stab, rtab, sidx2, ridx2)


def _sc_scatter(mji, ridx2, zer):
    mesh = plsc.VectorSubcoreMesh(core_axis_name="c", subcore_axis_name="s")

    @functools.partial(
        pl.kernel,
        out_type=jax.ShapeDtypeStruct((_NC, N, OUT), jnp.float32),
        mesh=mesh,
        scratch_types=[
            pltpu.VMEM((CPT, CHUNK), jnp.int32),
            pltpu.VMEM((2, GROWS, OUT), jnp.float32),
            pltpu.VMEM_SHARED((N, OUT), jnp.float32),
            pltpu.SemaphoreType.DMA,
            pltpu.SemaphoreType.DMA,
        ],
        compiler_params=pltpu.CompilerParams(use_tc_tiling_on_sc=False),
    )
    def k(mji_h, ridx_h, zer_h, pout_h, ridx_v, mbuf, acc, lsem0, lsem1):
        c = lax.axis_index("c")
        s = lax.axis_index("s")
        wid = c * _NS + s
        crow0 = wid * CPT
        erow0 = wid * EPT
        pltpu.sync_copy(ridx_h.at[pl.ds(crow0, CPT)], ridx_v)

        @pl.when(s == 0)
        def _():
            pltpu.sync_copy(zer_h, acc)

        plsc.subcore_barrier()

        lsems = (lsem0, lsem1)

        def start_load(g, p):
            pltpu.async_copy(mji_h.at[pl.ds(erow0 + g * GROWS, GROWS)],
                             mbuf.at[p], lsems[p])

        def drain_load(p):
            pltpu.make_async_copy(mji_h.at[pl.ds(0, GROWS)], mbuf.at[p], lsems[p]).wait()

        def scatter_group(g, p):
            for cc in range(GCH):
                j = g * GCH + cc
                pltpu.sync_copy(mbuf.at[p].at[pl.ds(cc * CHUNK, CHUNK)],
                                acc.at[ridx_v.at[j]], add=True)

        start_load(0, 0)

        def body(i, carry):
            g0 = 2 * i
            g1 = g0 + 1
            start_load(g1, 1)
            drain_load(0)
            scatter_group(g0, 0)

            @pl.when(i < (NG // 2 - 1))
            def _():
                start_load(g0 + 2, 0)

            drain_load(1)
            scatter_group(g1, 1)
            return carry

        lax.fori_loop(0, NG // 2, body, 0)
        plsc.subcore_barrier()

        @pl.when(s == 0)
        def _():
            pltpu.sync_copy(acc, pout_h.at[c])

    return k(mji, ridx2, zer)


def _tc_compute(ef, ea, gs, gr, w1e, w1s, w1r, w2p, r1s, s2):
    def body(ef_r, ea_r, gs_r, gr_r, w1e_r, w1s_r, w1r_r, w2_r, r1_r, s2_r, out_r):
        dot = functools.partial(jnp.dot, preferred_element_type=jnp.float32)
        hpre = (dot(ef_r[...], w1e_r[...])
                + dot(gs_r[...], w1s_r[...])
                + dot(gr_r[...], w1r_r[...]))
        h = jnp.maximum(hpre, 0.0)
        t = dot(h, w2_r[...])
        ea_v = ea_r[...]
        v = (ea_v[:, 0:1] * t[:, 0:128]
             + ea_v[:, 1:2] * t[:, 128:256]
             + ea_v[:, 2:3] * t[:, 256:384]
             + ea_v[:, 3:4] * t[:, 384:512])
        x1e = dot(gs_r[:, 0:16], r1_r[...])
        out_r[...] = dot(v * x1e, s2_r[...])

    be = lambda d: pl.BlockSpec((BE, d), lambda i: (i, 0))
    full = lambda a: pl.BlockSpec(a.shape, lambda i: (0,) * a.ndim)
    return pl.pallas_call(
        body,
        grid=(E // BE,),
        in_specs=[be(EDGE_FEAT), be(EDGE_ATTR), be(SROW), be(RROW),
                  full(w1e), full(w1s), full(w1r), full(w2p),
                  full(r1s), full(s2)],
        out_specs=be(OUT),
        out_shape=jax.ShapeDtypeStruct((E, OUT), jnp.float32),
    )(ef, ea, gs, gr, w1e, w1s, w1r, w2p, r1s, s2)


def _combine(p):
    def body(p_r, o_r):
        o_r[...] = p_r[0] + p_r[1]

    return pl.pallas_call(
        body,
        out_shape=jax.ShapeDtypeStruct((N, OUT), jnp.float32),
    )(p)


def kernel(node_attrs, node_feats, edge_attrs, edge_feats, edge_index, W1, W2, w_lin):
    f32 = jnp.float32
    inv = 1.0 / math.sqrt(float(MLP_IN))
    w1n = W1 * inv
    w1e = w1n[0:EDGE_FEAT]
    w1s = jnp.concatenate(
        [jnp.zeros((NODE_FEAT, MLP_IN), f32),
         w1n[EDGE_FEAT:EDGE_FEAT + NUM_ELEM],
         jnp.zeros((SROW - NODE_FEAT - NUM_ELEM, MLP_IN), f32)], axis=0)
    w1r = jnp.concatenate(
        [w1n[EDGE_FEAT + NUM_ELEM:],
         jnp.zeros((RROW - NUM_ELEM, MLP_IN), f32)], axis=0)
    w2n = W2 * (math.sqrt(2.0) * inv)
    w2p = w2n.reshape(MLP_IN, NODE_FEAT, EDGE_ATTR, OUT).transpose(0, 2, 1, 3) \
             .reshape(MLP_IN, NODE_FEAT * EDGE_ATTR * OUT)
    s2 = jnp.tile(w_lin, (NODE_FEAT, 1)) * (
        1.0 / (math.sqrt(float(NODE_FEAT * EDGE_ATTR)) * math.sqrt(float(OUT))))
    r1s = jnp.asarray(np.repeat(np.eye(NODE_FEAT, dtype=np.float32), OUT, axis=1))

    stab = jnp.concatenate([node_feats, node_attrs, jnp.zeros((N, SROW - NODE_FEAT - NUM_ELEM), f32)], axis=1)
    rtab = jnp.concatenate([node_attrs, jnp.zeros((N, RROW - NUM_ELEM), f32)], axis=1)

    sidx = edge_index[0].reshape(E // CHUNK, CHUNK)
    ridx = edge_index[1].reshape(E // CHUNK, CHUNK)

    gs, gr = _sc_gather(stab, rtab, sidx, ridx)
    mji = _tc_compute(edge_feats, edge_attrs, gs, gr, w1e, w1s, w1r, w2p, r1s, s2)
    zer = jnp.zeros((N, OUT), f32)
    p = _sc_scatter(mji, ridx, zer)
    return _combine(p)


# speedup vs baseline: 4.6101x; 1.2991x over previous
import functools
import math

import jax
import jax.numpy as jnp
import numpy as np
from jax import lax
from jax.experimental import pallas as pl
from jax.experimental.pallas import tpu as pltpu
from jax.experimental.pallas import tpu_sc as plsc

N = 10000
E = 160000
NUM_ELEM = 10
EDGE_FEAT = 16
NODE_FEAT = 16
EDGE_ATTR = 4
OUT = 8
MLP_IN = EDGE_FEAT + 2 * NUM_ELEM

_NC = 2
_NS = 16
_NW = _NC * _NS
CHUNK = 125
GCH = 5
NG = 8
GROWS = GCH * CHUNK
CPT = NG * GCH
EPT = CPT * CHUNK
SROW = 32
RROW = 16

BE = 2000


def _sc_gather(stab, rtab, sidx2, ridx2):
    mesh = plsc.VectorSubcoreMesh(core_axis_name="c", subcore_axis_name="s")

    @functools.partial(
        pl.kernel,
        out_type=jax.ShapeDtypeStruct((E, 128), jnp.float32),
        mesh=mesh,
        scratch_types=[
            pltpu.VMEM((CPT, CHUNK), jnp.int32),
            pltpu.VMEM((CPT, CHUNK), jnp.int32),
            pltpu.VMEM((2, GROWS, SROW), jnp.float32),
            pltpu.VMEM((2, GROWS, RROW), jnp.float32),
            pltpu.SemaphoreType.DMA,
            pltpu.SemaphoreType.DMA,
            pltpu.SemaphoreType.DMA,
            pltpu.SemaphoreType.DMA,
        ],
        compiler_params=pltpu.CompilerParams(use_tc_tiling_on_sc=False),
    )
    def k(stab_h, rtab_h, sidx_h, ridx_h, gc_h,
          sidx_v, ridx_v, sbuf, rbuf, gsem0, gsem1, wsem0, wsem1):
        wid = lax.axis_index("s") * _NC + lax.axis_index("c")
        crow0 = wid * CPT
        erow0 = wid * EPT
        pltpu.sync_copy(sidx_h.at[pl.ds(crow0, CPT)], sidx_v)
        pltpu.sync_copy(ridx_h.at[pl.ds(crow0, CPT)], ridx_v)

        gsems = (gsem0, gsem1)
        wsems = (wsem0, wsem1)

        def start_gathers(g, p):
            for c in range(GCH):
                j = g * GCH + c
                pltpu.async_copy(stab_h.at[sidx_v.at[j]],
                                 sbuf.at[p].at[pl.ds(c * CHUNK, CHUNK)], gsems[p])
                pltpu.async_copy(rtab_h.at[ridx_v.at[j]],
                                 rbuf.at[p].at[pl.ds(c * CHUNK, CHUNK)], gsems[p])

        def drain_gathers(p):
            pltpu.make_async_copy(stab_h.at[pl.ds(0, GROWS)], sbuf.at[p], gsems[p]).wait()
            pltpu.make_async_copy(rtab_h.at[pl.ds(0, GROWS)], rbuf.at[p], gsems[p]).wait()

        def start_wb(g, p):
            base = erow0 + g * GROWS
            pltpu.async_copy(sbuf.at[p],
                             gc_h.at[pl.ds(base, GROWS), pl.ds(0, SROW)], wsems[p])
            pltpu.async_copy(rbuf.at[p],
                             gc_h.at[pl.ds(base, GROWS), pl.ds(SROW, RROW)], wsems[p])

        def drain_wb(p):
            pltpu.make_async_copy(sbuf.at[p],
                                  gc_h.at[pl.ds(0, GROWS), pl.ds(0, SROW)], wsems[p]).wait()
            pltpu.make_async_copy(rbuf.at[p],
                                  gc_h.at[pl.ds(0, GROWS), pl.ds(SROW, RROW)], wsems[p]).wait()

        start_gathers(0, 0)

        def body(i, carry):
            g0 = 2 * i
            g1 = g0 + 1
            drain_gathers(0)
            start_wb(g0, 0)

            @pl.when(i > 0)
            def _():
                drain_wb(1)

            start_gathers(g1, 1)
            drain_gathers(1)
            start_wb(g1, 1)
            drain_wb(0)

            @pl.when(i < (NG // 2 - 1))
            def _():
                start_gathers(g0 + 2, 0)

            return carry

        lax.fori_loop(0, NG // 2, body, 0)
        drain_wb(1)

    return k(stab, rtab, sidx2, ridx2)


def _sc_scatter(mji, ridx2, zer):
    mesh = plsc.VectorSubcoreMesh(core_axis_name="c", subcore_axis_name="s")

    @functools.partial(
        pl.kernel,
        out_type=jax.ShapeDtypeStruct((_NC, N, OUT), jnp.float32),
        mesh=mesh,
        scratch_types=[
            pltpu.VMEM((CPT, CHUNK), jnp.int32),
            pltpu.VMEM((2, GROWS, OUT), jnp.float32),
            pltpu.VMEM_SHARED((N, OUT), jnp.float32),
            pltpu.SemaphoreType.DMA,
            pltpu.SemaphoreType.DMA,
        ],
        compiler_params=pltpu.CompilerParams(use_tc_tiling_on_sc=False),
    )
    def k(mji_h, ridx_h, zer_h, pout_h, ridx_v, mbuf, acc, lsem0, lsem1):
        c = lax.axis_index("c")
        s = lax.axis_index("s")
        wid = c * _NS + s
        crow0 = wid * CPT
        erow0 = wid * EPT
        pltpu.sync_copy(ridx_h.at[pl.ds(crow0, CPT)], ridx_v)

        @pl.when(s == 0)
        def _():
            pltpu.sync_copy(zer_h, acc)

        plsc.subcore_barrier()

        lsems = (lsem0, lsem1)

        def start_load(g, p):
            pltpu.async_copy(mji_h.at[pl.ds(erow0 + g * GROWS, GROWS)],
                             mbuf.at[p], lsems[p])

        def drain_load(p):
            pltpu.make_async_copy(mji_h.at[pl.ds(0, GROWS)], mbuf.at[p], lsems[p]).wait()

        def scatter_group(g, p):
            for cc in range(GCH):
                j = g * GCH + cc
                pltpu.sync_copy(mbuf.at[p].at[pl.ds(cc * CHUNK, CHUNK)],
                                acc.at[ridx_v.at[j]], add=True)

        start_load(0, 0)

        def body(i, carry):
            g0 = 2 * i
            g1 = g0 + 1
            start_load(g1, 1)
            drain_load(0)
            scatter_group(g0, 0)

            @pl.when(i < (NG // 2 - 1))
            def _():
                start_load(g0 + 2, 0)

            drain_load(1)
            scatter_group(g1, 1)
            return carry

        lax.fori_loop(0, NG // 2, body, 0)
        plsc.subcore_barrier()

        @pl.when(s == 0)
        def _():
            pltpu.sync_copy(acc, pout_h.at[c])

    return k(mji, ridx2, zer)


def _tc_compute(ef, ea, gc, w1e, w1c, w2p, r1s, s2):
    def body(ef_r, ea_r, gc_r, w1e_r, w1c_r, w2_r, r1_r, s2_r, out_r):
        dot = functools.partial(jnp.dot, preferred_element_type=jnp.float32)
        gc_v = gc_r[:, 0:SROW + RROW]
        hpre = dot(ef_r[...], w1e_r[...]) + dot(gc_v, w1c_r[...])
        h = jnp.maximum(hpre, 0.0)
        t = dot(h, w2_r[...])
        ea_v = ea_r[...]
        v = (ea_v[:, 0:1] * t[:, 0:128]
             + ea_v[:, 1:2] * t[:, 128:256]
             + ea_v[:, 2:3] * t[:, 256:384]
             + ea_v[:, 3:4] * t[:, 384:512])
        x1e = dot(gc_v[:, 0:16], r1_r[...])
        out_r[...] = dot(v * x1e, s2_r[...])

    be = lambda d: pl.BlockSpec((BE, d), lambda i: (i, 0))
    full = lambda a: pl.BlockSpec(a.shape, lambda i: (0,) * a.ndim)
    return pl.pallas_call(
        body,
        grid=(E // BE,),
        in_specs=[be(EDGE_FEAT), be(EDGE_ATTR), be(128),
                  full(w1e), full(w1c), full(w2p),
                  full(r1s), full(s2)],
        out_specs=be(OUT),
        out_shape=jax.ShapeDtypeStruct((E, OUT), jnp.float32),
    )(ef, ea, gc, w1e, w1c, w2p, r1s, s2)


def _combine(p):
    def body(p_r, o_r):
        o_r[...] = p_r[0] + p_r[1]

    return pl.pallas_call(
        body,
        out_shape=jax.ShapeDtypeStruct((N, OUT), jnp.float32),
    )(p)


def kernel(node_attrs, node_feats, edge_attrs, edge_feats, edge_index, W1, W2, w_lin):
    f32 = jnp.float32
    inv = 1.0 / math.sqrt(float(MLP_IN))
    w1n = W1 * inv
    w1e = w1n[0:EDGE_FEAT]
    w1c = jnp.concatenate(
        [jnp.zeros((NODE_FEAT, MLP_IN), f32),
         w1n[EDGE_FEAT:EDGE_FEAT + NUM_ELEM],
         jnp.zeros((SROW - NODE_FEAT - NUM_ELEM, MLP_IN), f32),
         w1n[EDGE_FEAT + NUM_ELEM:],
         jnp.zeros((RROW - NUM_ELEM, MLP_IN), f32)], axis=0)
    w2n = W2 * (math.sqrt(2.0) * inv)
    w2p = w2n.reshape(MLP_IN, NODE_FEAT, EDGE_ATTR, OUT).transpose(0, 2, 1, 3) \
             .reshape(MLP_IN, NODE_FEAT * EDGE_ATTR * OUT)
    s2 = jnp.tile(w_lin, (NODE_FEAT, 1)) * (
        1.0 / (math.sqrt(float(NODE_FEAT * EDGE_ATTR)) * math.sqrt(float(OUT))))
    r1s = jnp.asarray(np.repeat(np.eye(NODE_FEAT, dtype=np.float32), OUT, axis=1))

    stab = jnp.concatenate([node_feats, node_attrs, jnp.zeros((N, SROW - NODE_FEAT - NUM_ELEM), f32)], axis=1)
    rtab = jnp.concatenate([node_attrs, jnp.zeros((N, RROW - NUM_ELEM), f32)], axis=1)

    sidx = edge_index[0].reshape(E // CHUNK, CHUNK)
    ridx = edge_index[1].reshape(E // CHUNK, CHUNK)

    gc = _sc_gather(stab, rtab, sidx, ridx)
    mji = _tc_compute(edge_feats, edge_attrs, gc, w1e, w1c, w2p, r1s, s2)
    zer = jnp.zeros((N, OUT), f32)
    p = _sc_scatter(mji, ridx, zer)
    return _combine(p)


# speedup vs baseline: 6.5356x; 1.4177x over previous
import functools
import math

import jax
import jax.numpy as jnp
import numpy as np
from jax import lax
from jax.experimental import pallas as pl
from jax.experimental.pallas import tpu as pltpu
from jax.experimental.pallas import tpu_sc as plsc

N = 10000
E = 160000
NUM_ELEM = 10
EDGE_FEAT = 16
NODE_FEAT = 16
EDGE_ATTR = 4
OUT = 8
MLP_IN = EDGE_FEAT + 2 * NUM_ELEM

_NC = 2
_NS = 16
_NW = _NC * _NS
CHUNK = 125
GCH = 5
NG = 8
GROWS = GCH * CHUNK
CPT = NG * GCH
EPT = CPT * CHUNK
SROW = 32
RROW = 16

BE = 3200


def _sc_gather(stab, rtab, sidx2, ridx2):
    mesh = plsc.VectorSubcoreMesh(core_axis_name="c", subcore_axis_name="s")

    @functools.partial(
        pl.kernel,
        out_type=jax.ShapeDtypeStruct((E, 128), jnp.float32),
        mesh=mesh,
        scratch_types=[
            pltpu.VMEM((CPT, CHUNK), jnp.int32),
            pltpu.VMEM((CPT, CHUNK), jnp.int32),
            pltpu.VMEM((2, GROWS, SROW), jnp.float32),
            pltpu.VMEM((2, GROWS, RROW), jnp.float32),
            pltpu.SemaphoreType.DMA,
            pltpu.SemaphoreType.DMA,
            pltpu.SemaphoreType.DMA,
            pltpu.SemaphoreType.DMA,
        ],
        compiler_params=pltpu.CompilerParams(use_tc_tiling_on_sc=False),
    )
    def k(stab_h, rtab_h, sidx_h, ridx_h, gc_h,
          sidx_v, ridx_v, sbuf, rbuf, gsem0, gsem1, wsem0, wsem1):
        wid = lax.axis_index("s") * _NC + lax.axis_index("c")
        crow0 = wid * CPT
        erow0 = wid * EPT
        pltpu.sync_copy(sidx_h.at[pl.ds(crow0, CPT)], sidx_v)
        pltpu.sync_copy(ridx_h.at[pl.ds(crow0, CPT)], ridx_v)

        gsems = (gsem0, gsem1)
        wsems = (wsem0, wsem1)

        def start_gathers(g, p):
            for c in range(GCH):
                j = g * GCH + c
                pltpu.async_copy(stab_h.at[sidx_v.at[j]],
                                 sbuf.at[p].at[pl.ds(c * CHUNK, CHUNK)], gsems[p])
                pltpu.async_copy(rtab_h.at[ridx_v.at[j]],
                                 rbuf.at[p].at[pl.ds(c * CHUNK, CHUNK)], gsems[p])

        def drain_gathers(p):
            pltpu.make_async_copy(stab_h.at[pl.ds(0, GROWS)], sbuf.at[p], gsems[p]).wait()
            pltpu.make_async_copy(rtab_h.at[pl.ds(0, GROWS)], rbuf.at[p], gsems[p]).wait()

        def start_wb(g, p):
            base = erow0 + g * GROWS
            pltpu.async_copy(sbuf.at[p],
                             gc_h.at[pl.ds(base, GROWS), pl.ds(0, SROW)], wsems[p])
            pltpu.async_copy(rbuf.at[p],
                             gc_h.at[pl.ds(base, GROWS), pl.ds(SROW, RROW)], wsems[p])

        def drain_wb(p):
            pltpu.make_async_copy(sbuf.at[p],
                                  gc_h.at[pl.ds(0, GROWS), pl.ds(0, SROW)], wsems[p]).wait()
            pltpu.make_async_copy(rbuf.at[p],
                                  gc_h.at[pl.ds(0, GROWS), pl.ds(SROW, RROW)], wsems[p]).wait()

        start_gathers(0, 0)

        def body(i, carry):
            g0 = 2 * i
            g1 = g0 + 1
            drain_gathers(0)
            start_wb(g0, 0)

            @pl.when(i > 0)
            def _():
                drain_wb(1)

            start_gathers(g1, 1)
            drain_gathers(1)
            start_wb(g1, 1)
            drain_wb(0)

            @pl.when(i < (NG // 2 - 1))
            def _():
                start_gathers(g0 + 2, 0)

            return carry

        lax.fori_loop(0, NG // 2, body, 0)
        drain_wb(1)

    return k(stab, rtab, sidx2, ridx2)


def _sc_scatter(mji, ridx2, zer):
    mesh = plsc.VectorSubcoreMesh(core_axis_name="c", subcore_axis_name="s")

    @functools.partial(
        pl.kernel,
        out_type=jax.ShapeDtypeStruct((_NC, N, OUT), jnp.float32),
        mesh=mesh,
        scratch_types=[
            pltpu.VMEM((CPT, CHUNK), jnp.int32),
            pltpu.VMEM((2, GROWS, OUT), jnp.float32),
            pltpu.VMEM_SHARED((N, OUT), jnp.float32),
            pltpu.SemaphoreType.DMA,
            pltpu.SemaphoreType.DMA,
        ],
        compiler_params=pltpu.CompilerParams(use_tc_tiling_on_sc=False),
    )
    def k(mji_h, ridx_h, zer_h, pout_h, ridx_v, mbuf, acc, lsem0, lsem1):
        c = lax.axis_index("c")
        s = lax.axis_index("s")
        wid = c * _NS + s
        crow0 = wid * CPT
        erow0 = wid * EPT
        pltpu.sync_copy(ridx_h.at[pl.ds(crow0, CPT)], ridx_v)

        @pl.when(s == 0)
        def _():
            pltpu.sync_copy(zer_h, acc)

        plsc.subcore_barrier()

        lsems = (lsem0, lsem1)

        def start_load(g, p):
            pltpu.async_copy(mji_h.at[pl.ds(erow0 + g * GROWS, GROWS), pl.ds(0, OUT)],
                             mbuf.at[p], lsems[p])

        def drain_load(p):
            pltpu.make_async_copy(mji_h.at[pl.ds(0, GROWS), pl.ds(0, OUT)],
                                  mbuf.at[p], lsems[p]).wait()

        def scatter_group(g, p):
            for cc in range(GCH):
                j = g * GCH + cc
                pltpu.sync_copy(mbuf.at[p].at[pl.ds(cc * CHUNK, CHUNK)],
                                acc.at[ridx_v.at[j]], add=True)

        start_load(0, 0)

        def body(i, carry):
            g0 = 2 * i
            g1 = g0 + 1
            start_load(g1, 1)
            drain_load(0)
            scatter_group(g0, 0)

            @pl.when(i < (NG // 2 - 1))
            def _():
                start_load(g0 + 2, 0)

            drain_load(1)
            scatter_group(g1, 1)
            return carry

        lax.fori_loop(0, NG // 2, body, 0)
        plsc.subcore_barrier()

        @pl.when(s == 0)
        def _():
            pltpu.sync_copy(acc, pout_h.at[c])

    return k(mji, ridx2, zer)


def _tc_compute(efT, eaT, gc, w1e, w1c, w2p, r1s, s2):
    def body(ef_r, ea_r, gc_r, w1e_r, w1c_r, w2_r, r1_r, s2_r, out_r):
        dot = functools.partial(jnp.dot, preferred_element_type=jnp.float32)
        dg = functools.partial(lax.dot_general, preferred_element_type=jnp.float32)
        gc_v = gc_r[:, 0:SROW + RROW]
        hpre = (dg(ef_r[...], w1e_r[...], (((0,), (0,)), ((), ())))
                + dot(gc_v, w1c_r[...]))
        h = jnp.maximum(hpre, 0.0)
        t = dot(h.astype(jnp.bfloat16), w2_r[...])
        ea_v = dg(ea_r[...], jnp.eye(EDGE_ATTR, dtype=jnp.float32),
                  (((0,), (0,)), ((), ())))
        v = (ea_v[:, 0:1] * t[:, 0:128]
             + ea_v[:, 1:2] * t[:, 128:256]
             + ea_v[:, 2:3] * t[:, 256:384]
             + ea_v[:, 3:4] * t[:, 384:512])
        x1e = dot(gc_v[:, 0:16], r1_r[...])
        out_r[...] = dot(v * x1e, s2_r[...])

    be = lambda d: pl.BlockSpec((BE, d), lambda i: (i, 0))
    beT = lambda d: pl.BlockSpec((d, BE), lambda i: (0, i))
    full = lambda a: pl.BlockSpec(a.shape, lambda i: (0,) * a.ndim)
    return pl.pallas_call(
        body,
        grid=(E // BE,),
        in_specs=[beT(EDGE_FEAT), beT(EDGE_ATTR), be(128),
                  full(w1e), full(w1c), full(w2p),
                  full(r1s), full(s2)],
        out_specs=be(128),
        out_shape=jax.ShapeDtypeStruct((E, 128), jnp.float32),
    )(efT, eaT, gc, w1e, w1c, w2p, r1s, s2)


def _combine(p):
    def body(p_r, o_r):
        o_r[...] = p_r[0] + p_r[1]

    return pl.pallas_call(
        body,
        out_shape=jax.ShapeDtypeStruct((N, OUT), jnp.float32),
    )(p)


def kernel(node_attrs, node_feats, edge_attrs, edge_feats, edge_index, W1, W2, w_lin):
    f32 = jnp.float32
    inv = 1.0 / math.sqrt(float(MLP_IN))
    w1n = W1 * inv
    w1e = w1n[0:EDGE_FEAT]
    w1c = jnp.concatenate(
        [jnp.zeros((NODE_FEAT, MLP_IN), f32),
         w1n[EDGE_FEAT:EDGE_FEAT + NUM_ELEM],
         jnp.zeros((SROW - NODE_FEAT - NUM_ELEM, MLP_IN), f32),
         w1n[EDGE_FEAT + NUM_ELEM:],
         jnp.zeros((RROW - NUM_ELEM, MLP_IN), f32)], axis=0)
    w2n = W2 * (math.sqrt(2.0) * inv)
    w2p = w2n.reshape(MLP_IN, NODE_FEAT, EDGE_ATTR, OUT).transpose(0, 2, 1, 3) \
             .reshape(MLP_IN, NODE_FEAT * EDGE_ATTR * OUT).astype(jnp.bfloat16)
    s2 = jnp.tile(w_lin, (NODE_FEAT, 1)) * (
        1.0 / (math.sqrt(float(NODE_FEAT * EDGE_ATTR)) * math.sqrt(float(OUT))))
    s2 = jnp.pad(s2, ((0, 0), (0, 128 - OUT)))
    r1s = jnp.asarray(np.repeat(np.eye(NODE_FEAT, dtype=np.float32), OUT, axis=1))

    stab = jnp.concatenate([node_feats, node_attrs, jnp.zeros((N, SROW - NODE_FEAT - NUM_ELEM), f32)], axis=1)
    rtab = jnp.concatenate([node_attrs, jnp.zeros((N, RROW - NUM_ELEM), f32)], axis=1)

    sidx = edge_index[0].reshape(E // CHUNK, CHUNK)
    ridx = edge_index[1].reshape(E // CHUNK, CHUNK)

    gc = _sc_gather(stab, rtab, sidx, ridx)
    mji = _tc_compute(edge_feats.T, edge_attrs.T, gc, w1e, w1c, w2p, r1s, s2)
    zer = jnp.zeros((N, OUT), f32)
    p = _sc_scatter(mji, ridx, zer)
    return _combine(p)


# speedup vs baseline: 6.7172x; 1.0278x over previous
import functools
import math

import jax
import jax.numpy as jnp
import numpy as np
from jax import lax
from jax.experimental import pallas as pl
from jax.experimental.pallas import tpu as pltpu
from jax.experimental.pallas import tpu_sc as plsc

N = 10000
E = 160000
NUM_ELEM = 10
EDGE_FEAT = 16
NODE_FEAT = 16
EDGE_ATTR = 4
OUT = 8
MLP_IN = EDGE_FEAT + 2 * NUM_ELEM

_NC = 2
_NS = 16
_NW = _NC * _NS
CHUNK = 125
GCH = 5
GROWS = GCH * CHUNK
NHALF = 2
EH = E // NHALF
NGH = EH // (_NW * GROWS)
CPTH = NGH * GCH
EPTH = CPTH * CHUNK
CROWS_H = EH // CHUNK
SROW = 32
RROW = 16

BE = 3200


def _sc_gather(stab, rtab, sidx2, ridx2, half):
    mesh = plsc.VectorSubcoreMesh(core_axis_name="c", subcore_axis_name="s")

    @functools.partial(
        pl.kernel,
        out_type=jax.ShapeDtypeStruct((EH, 128), jnp.float32),
        mesh=mesh,
        scratch_types=[
            pltpu.VMEM((CPTH, CHUNK), jnp.int32),
            pltpu.VMEM((CPTH, CHUNK), jnp.int32),
            pltpu.VMEM((2, GROWS, SROW), jnp.float32),
            pltpu.VMEM((2, GROWS, RROW), jnp.float32),
            pltpu.SemaphoreType.DMA,
            pltpu.SemaphoreType.DMA,
            pltpu.SemaphoreType.DMA,
            pltpu.SemaphoreType.DMA,
        ],
        compiler_params=pltpu.CompilerParams(use_tc_tiling_on_sc=False),
        name=f"gather_half{half}",
    )
    def k(stab_h, rtab_h, sidx_h, ridx_h, gc_h,
          sidx_v, ridx_v, sbuf, rbuf, gsem0, gsem1, wsem0, wsem1):
        wid = lax.axis_index("s") * _NC + lax.axis_index("c")
        crow0 = half * CROWS_H + wid * CPTH
        erow0 = wid * EPTH
        pltpu.sync_copy(sidx_h.at[pl.ds(crow0, CPTH)], sidx_v)
        pltpu.sync_copy(ridx_h.at[pl.ds(crow0, CPTH)], ridx_v)

        gsems = (gsem0, gsem1)
        wsems = (wsem0, wsem1)

        def start_gathers(g, p):
            for c in range(GCH):
                j = g * GCH + c
                pltpu.async_copy(stab_h.at[sidx_v.at[j]],
                                 sbuf.at[p].at[pl.ds(c * CHUNK, CHUNK)], gsems[p])
                pltpu.async_copy(rtab_h.at[ridx_v.at[j]],
                                 rbuf.at[p].at[pl.ds(c * CHUNK, CHUNK)], gsems[p])

        def drain_gathers(p):
            pltpu.make_async_copy(stab_h.at[pl.ds(0, GROWS)], sbuf.at[p], gsems[p]).wait()
            pltpu.make_async_copy(rtab_h.at[pl.ds(0, GROWS)], rbuf.at[p], gsems[p]).wait()

        def start_wb(g, p):
            base = erow0 + g * GROWS
            pltpu.async_copy(sbuf.at[p],
                             gc_h.at[pl.ds(base, GROWS), pl.ds(0, SROW)], wsems[p])
            pltpu.async_copy(rbuf.at[p],
                             gc_h.at[pl.ds(base, GROWS), pl.ds(SROW, RROW)], wsems[p])

        def drain_wb(p):
            pltpu.make_async_copy(sbuf.at[p],
                                  gc_h.at[pl.ds(0, GROWS), pl.ds(0, SROW)], wsems[p]).wait()
            pltpu.make_async_copy(rbuf.at[p],
                                  gc_h.at[pl.ds(0, GROWS), pl.ds(SROW, RROW)], wsems[p]).wait()

        start_gathers(0, 0)

        def body(i, carry):
            g0 = 2 * i
            g1 = g0 + 1
            drain_gathers(0)
            start_wb(g0, 0)

            @pl.when(i > 0)
            def _():
                drain_wb(1)

            start_gathers(g1, 1)
            drain_gathers(1)
            start_wb(g1, 1)
            drain_wb(0)

            @pl.when(i < (NGH // 2 - 1))
            def _():
                start_gathers(g0 + 2, 0)

            return carry

        lax.fori_loop(0, NGH // 2, body, 0)
        drain_wb(1)

    return k(stab, rtab, sidx2, ridx2)


def _sc_scatter(mji0, mji1, ridx2, zer):
    mesh = plsc.VectorSubcoreMesh(core_axis_name="c", subcore_axis_name="s")

    @functools.partial(
        pl.kernel,
        out_type=jax.ShapeDtypeStruct((_NC, N, OUT), jnp.float32),
        mesh=mesh,
        scratch_types=[
            pltpu.VMEM((2 * CPTH, CHUNK), jnp.int32),
            pltpu.VMEM((2, GROWS, OUT), jnp.float32),
            pltpu.VMEM_SHARED((N, OUT), jnp.float32),
            pltpu.SemaphoreType.DMA,
            pltpu.SemaphoreType.DMA,
        ],
        compiler_params=pltpu.CompilerParams(use_tc_tiling_on_sc=False),
    )
    def k(mji0_h, mji1_h, ridx_h, zer_h, pout_h, ridx_v, mbuf, acc, lsem0, lsem1):
        c = lax.axis_index("c")
        s = lax.axis_index("s")
        wid = c * _NS + s
        erow0 = wid * EPTH
        pltpu.sync_copy(ridx_h.at[pl.ds(wid * CPTH, CPTH)],
                        ridx_v.at[pl.ds(0, CPTH)])
        pltpu.sync_copy(ridx_h.at[pl.ds(CROWS_H + wid * CPTH, CPTH)],
                        ridx_v.at[pl.ds(CPTH, CPTH)])

        @pl.when(s == 0)
        def _():
            pltpu.sync_copy(zer_h, acc)

        plsc.subcore_barrier()

        lsems = (lsem0, lsem1)

        def process_half(mji_h, jbase):
            def start_load(g, p):
                pltpu.async_copy(
                    mji_h.at[pl.ds(erow0 + g * GROWS, GROWS), pl.ds(0, OUT)],
                    mbuf.at[p], lsems[p])

            def drain_load(p):
                pltpu.make_async_copy(mji_h.at[pl.ds(0, GROWS), pl.ds(0, OUT)],
                                      mbuf.at[p], lsems[p]).wait()

            def scatter_group(g, p):
                for cc in range(GCH):
                    j = jbase + g * GCH + cc
                    pltpu.sync_copy(mbuf.at[p].at[pl.ds(cc * CHUNK, CHUNK)],
                                    acc.at[ridx_v.at[j]], add=True)

            start_load(0, 0)

            def body(i, carry):
                g0 = 2 * i
                g1 = g0 + 1
                start_load(g1, 1)
                drain_load(0)
                scatter_group(g0, 0)

                @pl.when(i < (NGH // 2 - 1))
                def _():
                    start_load(g0 + 2, 0)

                drain_load(1)
                scatter_group(g1, 1)
                return carry

            lax.fori_loop(0, NGH // 2, body, 0)

        process_half(mji0_h, 0)
        process_half(mji1_h, CPTH)
        plsc.subcore_barrier()

        @pl.when(s == 0)
        def _():
            pltpu.sync_copy(acc, pout_h.at[c])

    return k(mji0, mji1, ridx2, zer)


def _tc_compute(efT, eaT, gc, w1e, w1c, w2p, r1s, r2w, s2, half):
    def body(ef_r, ea_r, gc_r, w1e_r, w1c_r, w2_r, r1_r, r2w_r, s2_r, out_r):
        dot = functools.partial(jnp.dot, preferred_element_type=jnp.float32)
        dg = functools.partial(lax.dot_general, preferred_element_type=jnp.float32)
        gc_v = gc_r[:, 0:SROW + RROW]
        hpre = (dg(ef_r[...], w1e_r[...], (((0,), (0,)), ((), ())))
                + dot(gc_v, w1c_r[...]))
        h = jnp.maximum(hpre, 0.0)
        t = dot(h, w2_r[...])
        eaexp = dg(ea_r[...], r2w_r[...], (((0,), (0,)), ((), ())))
        u = t * eaexp
        v = u[:, 0:128] + u[:, 128:256] + u[:, 256:384] + u[:, 384:512]
        x1e = dot(gc_v[:, 0:16], r1_r[...])
        out_r[...] = dot(v * x1e, s2_r[...])

    off = half * (EH // BE)
    be = lambda d: pl.BlockSpec((BE, d), lambda i: (i, 0))
    beT = lambda d: pl.BlockSpec((d, BE), lambda i: (0, i + off))
    full = lambda a: pl.BlockSpec(a.shape, lambda i: (0,) * a.ndim)
    return pl.pallas_call(
        body,
        grid=(EH // BE,),
        in_specs=[beT(EDGE_FEAT), beT(EDGE_ATTR), be(128),
                  full(w1e), full(w1c), full(w2p),
                  full(r1s), full(r2w), full(s2)],
        out_specs=be(128),
        out_shape=jax.ShapeDtypeStruct((EH, 128), jnp.float32),
        name=f"tc_compute_half{half}",
    )(efT, eaT, gc, w1e, w1c, w2p, r1s, r2w, s2)


def _combine(p):
    def body(p_r, o_r):
        o_r[...] = p_r[0] + p_r[1]

    return pl.pallas_call(
        body,
        out_shape=jax.ShapeDtypeStruct((N, OUT), jnp.float32),
    )(p)


def kernel(node_attrs, node_feats, edge_attrs, edge_feats, edge_index, W1, W2, w_lin):
    f32 = jnp.float32
    inv = 1.0 / math.sqrt(float(MLP_IN))
    w1n = W1 * inv
    w1e = w1n[0:EDGE_FEAT]
    w1c = jnp.concatenate(
        [jnp.zeros((NODE_FEAT, MLP_IN), f32),
         w1n[EDGE_FEAT:EDGE_FEAT + NUM_ELEM],
         jnp.zeros((SROW - NODE_FEAT - NUM_ELEM, MLP_IN), f32),
         w1n[EDGE_FEAT + NUM_ELEM:],
         jnp.zeros((RROW - NUM_ELEM, MLP_IN), f32)], axis=0)
    w2n = W2 * (math.sqrt(2.0) * inv)
    w2p = w2n.reshape(MLP_IN, NODE_FEAT, EDGE_ATTR, OUT).transpose(0, 2, 1, 3) \
             .reshape(MLP_IN, NODE_FEAT * EDGE_ATTR * OUT)
    s2 = jnp.tile(w_lin, (NODE_FEAT, 1)) * (
        1.0 / (math.sqrt(float(NODE_FEAT * EDGE_ATTR)) * math.sqrt(float(OUT))))
    s2 = jnp.pad(s2, ((0, 0), (0, 128 - OUT)))
    r1s = jnp.asarray(np.repeat(np.eye(NODE_FEAT, dtype=np.float32), OUT, axis=1))
    r2w = jnp.asarray(np.repeat(np.eye(EDGE_ATTR, dtype=np.float32), 128, axis=1))

    stab = jnp.concatenate([node_feats, node_attrs,
                            jnp.zeros((N, SROW - NODE_FEAT - NUM_ELEM), f32)], axis=1)
    rtab = jnp.concatenate([node_attrs, jnp.zeros((N, RROW - NUM_ELEM), f32)], axis=1)

    sidx = edge_index[0].reshape(E // CHUNK, CHUNK)
    ridx = edge_index[1].reshape(E // CHUNK, CHUNK)
    efT = edge_feats.T
    eaT = edge_attrs.T

    gc0 = _sc_gather(stab, rtab, sidx, ridx, 0)
    mji0 = _tc_compute(efT, eaT, gc0, w1e, w1c, w2p, r1s, r2w, s2, 0)
    gc1 = _sc_gather(stab, rtab, sidx, ridx, 1)
    mji1 = _tc_compute(efT, eaT, gc1, w1e, w1c, w2p, r1s, r2w, s2, 1)

    zer = jnp.zeros((N, OUT), f32)
    p = _sc_scatter(mji0, mji1, ridx, zer)
    return _combine(p)


# speedup vs baseline: 8.4684x; 1.2607x over previous
import functools
import math

import jax
import jax.numpy as jnp
import numpy as np
from jax import lax
from jax.experimental import pallas as pl
from jax.experimental.pallas import tpu as pltpu
from jax.experimental.pallas import tpu_sc as plsc

N = 10000
E = 160000
NUM_ELEM = 10
EDGE_FEAT = 16
NODE_FEAT = 16
EDGE_ATTR = 4
OUT = 8
MLP_IN = EDGE_FEAT + 2 * NUM_ELEM

_NC = 2
_NS = 16
_NW = _NC * _NS
CHUNK = 125
GCH = 5
GROWS = GCH * CHUNK
NHALF = 2
EH = E // NHALF
NGH = EH // (_NW * GROWS)
CPTH = NGH * GCH
EPTH = CPTH * CHUNK
CROWS_H = EH // CHUNK
SROW = 32
RROW = 16

BE = 3200


def _sc_gather(stab, rtab, sidx2, ridx2, half):
    mesh = plsc.VectorSubcoreMesh(core_axis_name="c", subcore_axis_name="s")

    @functools.partial(
        pl.kernel,
        out_type=jax.ShapeDtypeStruct((EH, 128), jnp.float32),
        mesh=mesh,
        scratch_types=[
            pltpu.VMEM((CPTH, CHUNK), jnp.int32),
            pltpu.VMEM((CPTH, CHUNK), jnp.int32),
            pltpu.VMEM((2, GROWS, SROW), jnp.float32),
            pltpu.VMEM((2, GROWS, RROW), jnp.float32),
            pltpu.SemaphoreType.DMA,
            pltpu.SemaphoreType.DMA,
            pltpu.SemaphoreType.DMA,
            pltpu.SemaphoreType.DMA,
        ],
        compiler_params=pltpu.CompilerParams(use_tc_tiling_on_sc=False),
        name=f"gather_half{half}",
    )
    def k(stab_h, rtab_h, sidx_h, ridx_h, gc_h,
          sidx_v, ridx_v, sbuf, rbuf, gsem0, gsem1, wsem0, wsem1):
        wid = lax.axis_index("s") * _NC + lax.axis_index("c")
        crow0 = half * CROWS_H + wid * CPTH
        erow0 = wid * EPTH
        pltpu.sync_copy(sidx_h.at[pl.ds(crow0, CPTH)], sidx_v)
        pltpu.sync_copy(ridx_h.at[pl.ds(crow0, CPTH)], ridx_v)

        gsems = (gsem0, gsem1)
        wsems = (wsem0, wsem1)

        def start_gathers(g, p):
            for c in range(GCH):
                j = g * GCH + c
                pltpu.async_copy(stab_h.at[sidx_v.at[j]],
                                 sbuf.at[p].at[pl.ds(c * CHUNK, CHUNK)], gsems[p])
                pltpu.async_copy(rtab_h.at[ridx_v.at[j]],
                                 rbuf.at[p].at[pl.ds(c * CHUNK, CHUNK)], gsems[p])

        def drain_gathers(p):
            pltpu.make_async_copy(stab_h.at[pl.ds(0, GROWS)], sbuf.at[p], gsems[p]).wait()
            pltpu.make_async_copy(rtab_h.at[pl.ds(0, GROWS)], rbuf.at[p], gsems[p]).wait()

        def start_wb(g, p):
            base = erow0 + g * GROWS
            pltpu.async_copy(sbuf.at[p],
                             gc_h.at[pl.ds(base, GROWS), pl.ds(0, SROW)], wsems[p])
            pltpu.async_copy(rbuf.at[p],
                             gc_h.at[pl.ds(base, GROWS), pl.ds(SROW, RROW)], wsems[p])

        def drain_wb(p):
            pltpu.make_async_copy(sbuf.at[p],
                                  gc_h.at[pl.ds(0, GROWS), pl.ds(0, SROW)], wsems[p]).wait()
            pltpu.make_async_copy(rbuf.at[p],
                                  gc_h.at[pl.ds(0, GROWS), pl.ds(SROW, RROW)], wsems[p]).wait()

        start_gathers(0, 0)

        def body(i, carry):
            g0 = 2 * i
            g1 = g0 + 1
            drain_gathers(0)
            start_wb(g0, 0)

            @pl.when(i > 0)
            def _():
                drain_wb(1)

            start_gathers(g1, 1)
            drain_gathers(1)
            start_wb(g1, 1)
            drain_wb(0)

            @pl.when(i < (NGH // 2 - 1))
            def _():
                start_gathers(g0 + 2, 0)

            return carry

        lax.fori_loop(0, NGH // 2, body, 0)
        drain_wb(1)

    return k(stab, rtab, sidx2, ridx2)


def _sc_scatter(mji0, mji1, ridx2, zer):
    mesh = plsc.VectorSubcoreMesh(core_axis_name="c", subcore_axis_name="s")

    @functools.partial(
        pl.kernel,
        out_type=jax.ShapeDtypeStruct((_NC, N, OUT), jnp.float32),
        mesh=mesh,
        scratch_types=[
            pltpu.VMEM((2 * CPTH, CHUNK), jnp.int32),
            pltpu.VMEM((2, GROWS, OUT), jnp.float32),
            pltpu.VMEM_SHARED((N, OUT), jnp.float32),
            pltpu.SemaphoreType.DMA,
            pltpu.SemaphoreType.DMA,
        ],
        compiler_params=pltpu.CompilerParams(use_tc_tiling_on_sc=False),
    )
    def k(mji0_h, mji1_h, ridx_h, zer_h, pout_h, ridx_v, mbuf, acc, lsem0, lsem1):
        c = lax.axis_index("c")
        s = lax.axis_index("s")
        wid = c * _NS + s
        erow0 = wid * EPTH
        pltpu.sync_copy(ridx_h.at[pl.ds(wid * CPTH, CPTH)],
                        ridx_v.at[pl.ds(0, CPTH)])
        pltpu.sync_copy(ridx_h.at[pl.ds(CROWS_H + wid * CPTH, CPTH)],
                        ridx_v.at[pl.ds(CPTH, CPTH)])

        @pl.when(s == 0)
        def _():
            pltpu.sync_copy(zer_h, acc)

        plsc.subcore_barrier()

        lsems = (lsem0, lsem1)

        def process_half(mji_h, jbase):
            def start_load(g, p):
                pltpu.async_copy(
                    mji_h.at[pl.ds(erow0 + g * GROWS, GROWS), pl.ds(0, OUT)],
                    mbuf.at[p], lsems[p])

            def drain_load(p):
                pltpu.make_async_copy(mji_h.at[pl.ds(0, GROWS), pl.ds(0, OUT)],
                                      mbuf.at[p], lsems[p]).wait()

            def scatter_group(g, p):
                for cc in range(GCH):
                    j = jbase + g * GCH + cc
                    pltpu.sync_copy(mbuf.at[p].at[pl.ds(cc * CHUNK, CHUNK)],
                                    acc.at[ridx_v.at[j]], add=True)

            start_load(0, 0)

            def body(i, carry):
                g0 = 2 * i
                g1 = g0 + 1
                start_load(g1, 1)
                drain_load(0)
                scatter_group(g0, 0)

                @pl.when(i < (NGH // 2 - 1))
                def _():
                    start_load(g0 + 2, 0)

                drain_load(1)
                scatter_group(g1, 1)
                return carry

            lax.fori_loop(0, NGH // 2, body, 0)

        process_half(mji0_h, 0)
        process_half(mji1_h, CPTH)
        plsc.subcore_barrier()

        @pl.when(s == 0)
        def _():
            pltpu.sync_copy(acc, pout_h.at[c])

    return k(mji0, mji1, ridx2, zer)


def _tc_compute(efT, eaT, gc, w1e, w1c, w2p, r1s, r2w, s2, half):
    def body(ef_r, ea_r, gc_r, w1e_r, w1c_r, w2_r, r1_r, r2w_r, s2_r, out_r):
        dot = functools.partial(jnp.dot, preferred_element_type=jnp.float32)
        dg = functools.partial(lax.dot_general, preferred_element_type=jnp.float32)
        gc_v = gc_r[:, 0:SROW + RROW]
        hpre = (dg(ef_r[...], w1e_r[...], (((0,), (0,)), ((), ())))
                + dot(gc_v, w1c_r[...]))
        h2 = jnp.maximum(hpre, 0.0)
        eab = dg(ea_r[...], r2w_r[...], (((0,), (0,)), ((), ())))
        q = jnp.concatenate([h2 * eab[:, 0:128], h2 * eab[:, 128:256]], axis=1)
        v = dot(q, w2_r[...])
        x1e = dot(gc_v[:, 0:16], r1_r[...])
        out_r[...] = dot(v * x1e, s2_r[...])

    off = half * (EH // BE)
    be = lambda d: pl.BlockSpec((BE, d), lambda i: (i, 0))
    beT = lambda d: pl.BlockSpec((d, BE), lambda i: (0, i + off))
    full = lambda a: pl.BlockSpec(a.shape, lambda i: (0,) * a.ndim)
    return pl.pallas_call(
        body,
        grid=(EH // BE,),
        in_specs=[beT(EDGE_FEAT), beT(EDGE_ATTR), be(128),
                  full(w1e), full(w1c), full(w2p),
                  full(r1s), full(r2w), full(s2)],
        out_specs=be(128),
        out_shape=jax.ShapeDtypeStruct((EH, 128), jnp.float32),
        name=f"tc_compute_half{half}",
    )(efT, eaT, gc, w1e, w1c, w2p, r1s, r2w, s2)


def _combine(p):
    def body(p_r, o_r):
        o_r[...] = p_r[0] + p_r[1]

    return pl.pallas_call(
        body,
        out_shape=jax.ShapeDtypeStruct((N, OUT), jnp.float32),
    )(p)


def kernel(node_attrs, node_feats, edge_attrs, edge_feats, edge_index, W1, W2, w_lin):
    f32 = jnp.float32
    inv = 1.0 / math.sqrt(float(MLP_IN))
    w1n = W1 * inv
    w1e = w1n[0:EDGE_FEAT]
    w1c = jnp.concatenate(
        [jnp.zeros((NODE_FEAT, MLP_IN), f32),
         w1n[EDGE_FEAT:EDGE_FEAT + NUM_ELEM],
         jnp.zeros((SROW - NODE_FEAT - NUM_ELEM, MLP_IN), f32),
         w1n[EDGE_FEAT + NUM_ELEM:],
         jnp.zeros((RROW - NUM_ELEM, MLP_IN), f32)], axis=0)
    dup_np = np.zeros((MLP_IN, 128), np.float32)
    dup_np[np.arange(MLP_IN), np.arange(MLP_IN)] = 1.0
    dup_np[np.arange(MLP_IN), 40 + np.arange(MLP_IN)] = 1.0
    dupc = jnp.asarray(dup_np)
    w1e = w1e @ dupc
    w1c = w1c @ dupc
    w2n = W2 * (math.sqrt(2.0) * inv)
    w2j = w2n.reshape(MLP_IN, NODE_FEAT, EDGE_ATTR, OUT).transpose(2, 0, 1, 3) \
             .reshape(EDGE_ATTR, MLP_IN, NODE_FEAT * OUT)
    w2r = (jnp.zeros((256, NODE_FEAT * OUT), f32)
           .at[0:MLP_IN].set(w2j[0]).at[40:40 + MLP_IN].set(w2j[1])
           .at[128:128 + MLP_IN].set(w2j[2]).at[168:168 + MLP_IN].set(w2j[3]))
    s2 = jnp.tile(w_lin, (NODE_FEAT, 1)) * (
        1.0 / (math.sqrt(float(NODE_FEAT * EDGE_ATTR)) * math.sqrt(float(OUT))))
    s2 = jnp.pad(s2, ((0, 0), (0, 128 - OUT)))
    r1s = jnp.asarray(np.repeat(np.eye(NODE_FEAT, dtype=np.float32), OUT, axis=1))
    reab_np = np.zeros((EDGE_ATTR, 256), np.float32)
    reab_np[0, 0:MLP_IN] = 1.0
    reab_np[1, 40:40 + MLP_IN] = 1.0
    reab_np[2, 128:128 + MLP_IN] = 1.0
    reab_np[3, 168:168 + MLP_IN] = 1.0
    reab = jnp.asarray(reab_np)

    stab = jnp.concatenate([node_feats, node_attrs,
                            jnp.zeros((N, SROW - NODE_FEAT - NUM_ELEM), f32)], axis=1)
    rtab = jnp.concatenate([node_attrs, jnp.zeros((N, RROW - NUM_ELEM), f32)], axis=1)

    sidx = edge_index[0].reshape(E // CHUNK, CHUNK)
    ridx = edge_index[1].reshape(E // CHUNK, CHUNK)
    efT = edge_feats.T
    eaT = edge_attrs.T

    gc0 = _sc_gather(stab, rtab, sidx, ridx, 0)
    mji0 = _tc_compute(efT, eaT, gc0, w1e, w1c, w2r, r1s, reab, s2, 0)
    gc1 = _sc_gather(stab, rtab, sidx, ridx, 1)
    mji1 = _tc_compute(efT, eaT, gc1, w1e, w1c, w2r, r1s, reab, s2, 1)

    zer = jnp.zeros((N, OUT), f32)
    p = _sc_scatter(mji0, mji1, ridx, zer)
    return _combine(p)
